# baseline (speedup 1.0000x reference)
import jax
import jax.numpy as jnp
from jax.experimental import pallas as pl
from jax.experimental.pallas import tpu as pltpu

_N, _L, _C = 16, 4096, 64
_BN = 8


def _body(p_ref, l_ref, out_ref, acc_ref):
    step = pl.program_id(0)

    @pl.when(step == 0)
    def _init():
        acc_ref[0] = 0.0
        acc_ref[1] = 0.0

    l = l_ref[...]
    p = p_ref[...]
    nan = jnp.isnan(l)
    d = jnp.where(nan, 0.0, p - l)
    acc_ref[0] += jnp.sum(d * d)
    acc_ref[1] += float(_C) * jnp.sum(jnp.where(nan[:, :1, :], 0.0, 1.0))

    @pl.when(step == pl.num_programs(0) - 1)
    def _fin():
        out_ref[0] = acc_ref[0] / acc_ref[1]


def kernel(preds, labels):
    pt = preds.swapaxes(1, 2)
    lt = labels.swapaxes(1, 2)
    out = pl.pallas_call(
        _body,
        grid=(_N // _BN,),
        in_specs=[
            pl.BlockSpec((_BN, _C, _L), lambda i: (i, 0, 0)),
            pl.BlockSpec((_BN, _C, _L), lambda i: (i, 0, 0)),
        ],
        out_specs=pl.BlockSpec(memory_space=pltpu.SMEM),
        out_shape=jax.ShapeDtypeStruct((1,), jnp.float32),
        scratch_shapes=[pltpu.SMEM((2,), jnp.float32)],
    )(pt, lt)
    return out[0]


# speedup vs baseline: 1.1245x; 1.1245x over previous
import jax
import jax.numpy as jnp
from jax.experimental import pallas as pl
from jax.experimental.pallas import tpu as pltpu

_N, _L, _C = 16, 4096, 64
_BN = 4


def _body(p_ref, l_ref, out_ref, acc_ref):
    step = pl.program_id(0)

    @pl.when(step == 0)
    def _init():
        acc_ref[0] = 0.0
        acc_ref[1] = 0.0

    l = l_ref[...]
    p = p_ref[...]
    nan = jnp.isnan(l)
    d = jnp.where(nan, 0.0, p - l)
    acc_ref[0] += jnp.sum(d * d)
    acc_ref[1] += jnp.sum(jnp.where(nan, 0.0, 1.0))

    @pl.when(step == pl.num_programs(0) - 1)
    def _fin():
        out_ref[0] = acc_ref[0] / acc_ref[1]


def kernel(preds, labels):
    pt = preds.swapaxes(1, 2)
    lt = labels.swapaxes(1, 2)
    out = pl.pallas_call(
        _body,
        grid=(_N // _BN,),
        in_specs=[
            pl.BlockSpec((_BN, _C, _L), lambda i: (i, 0, 0)),
            pl.BlockSpec((_BN, _C, _L), lambda i: (i, 0, 0)),
        ],
        out_specs=pl.BlockSpec(memory_space=pltpu.SMEM),
        out_shape=jax.ShapeDtypeStruct((1,), jnp.float32),
        scratch_shapes=[pltpu.SMEM((2,), jnp.float32)],
    )(pt, lt)
    return out[0]
